# Initial kernel scaffold; baseline (speedup 1.0000x reference)
import jax
import jax.numpy as jnp
from jax.experimental import pallas as pl


def kernel(input_ids, position_ids, vocab_table, pos_table):
    raise NotImplementedError("write your pallas kernel here")


# speedup vs baseline: 1.6591x; 1.6591x over previous
import functools

import jax
import jax.numpy as jnp
from jax import lax
from jax.experimental import pallas as pl
from jax.experimental.pallas import tpu as pltpu
from jax.experimental.pallas import tpu_sc as plsc

VOCAB = 100000
DIM = 128
SEQ = 2048
BATCH = 4

N = BATCH * SEQ
NC = 2
NS = 16
NW = NC * NS
RPW = N // NW
CH = 128
NCH = RPW // CH
LANES = 16


_mesh = plsc.VectorSubcoreMesh(core_axis_name="c", subcore_axis_name="s")


@functools.partial(
    pl.kernel,
    mesh=_mesh,
    out_type=jax.ShapeDtypeStruct((N, DIM), jnp.float32),
    scratch_types=[
        pltpu.VMEM((NCH, CH), jnp.int32),
        pltpu.VMEM((NCH, CH), jnp.int32),
        pltpu.VMEM((RPW, DIM), jnp.float32),
        pltpu.VMEM((RPW, DIM), jnp.float32),
        pltpu.SemaphoreType.DMA,
    ],
)
def _emb_kernel(ids_hbm, pids_hbm, vocab_hbm, pos_hbm, out_hbm,
                idx_v, pidx_v, rows_v, prows_v, sem):
    wid = lax.axis_index("s") * NC + lax.axis_index("c")
    base = wid * RPW

    pltpu.sync_copy(ids_hbm.at[pl.ds(wid * NCH, NCH)], idx_v)
    pltpu.sync_copy(pids_hbm.at[pl.ds(wid * NCH, NCH)], pidx_v)

    copies = []
    for j in range(NCH):
        copies.append(pltpu.async_copy(
            vocab_hbm.at[idx_v.at[j]], rows_v.at[pl.ds(j * CH, CH)], sem))
        copies.append(pltpu.async_copy(
            pos_hbm.at[pidx_v.at[j]], prows_v.at[pl.ds(j * CH, CH)], sem))
    for cp in copies:
        cp.wait()

    def body(r, carry):
        for c in range(DIM // LANES):
            s = pl.ds(c * LANES, LANES)
            rows_v[r, s] = rows_v[r, s] + prows_v[r, s]
        return carry

    lax.fori_loop(0, RPW, body, 0)

    pltpu.sync_copy(rows_v, out_hbm.at[pl.ds(base, RPW)])


def kernel(input_ids, position_ids, vocab_table, pos_table):
    ids = input_ids.reshape(NW * NCH, CH).astype(jnp.int32)
    pids = position_ids.reshape(NW * NCH, CH).astype(jnp.int32)
    out = _emb_kernel(ids, pids, vocab_table, pos_table)
    return out.reshape(BATCH, SEQ, DIM)
